# baseline (speedup 1.0000x reference)
import functools

import jax
import jax.numpy as jnp
from jax import lax
from jax.experimental import pallas as pl
from jax.experimental.pallas import tpu as pltpu
from jax.experimental.pallas import tpu_sc as plsc

NUM_VIRTUAL_TOKENS = 20
TOKEN_DIM = 768
NUM_LAYERS = 12
EMBED_DIM = 2 * NUM_LAYERS * TOKEN_DIM
BATCH = 128

_INFO = plsc.get_sparse_core_info()
_NC = _INFO.num_cores
_NS = _INFO.num_subcores
_NW = _NC * _NS

_B = BATCH * NUM_VIRTUAL_TOKENS
_NG = 2
_NCH = _NW // _NG
_ROWS_PER_G = _B // _NG
_DC = EMBED_DIM // _NCH

@functools.partial(
    pl.kernel,
    mesh=plsc.VectorSubcoreMesh(core_axis_name="c", subcore_axis_name="s"),
    out_type=jax.ShapeDtypeStruct((_B, EMBED_DIM), jnp.float32),
    scratch_types=[
        pltpu.VMEM((_ROWS_PER_G,), jnp.int32),
        pltpu.VMEM((NUM_VIRTUAL_TOKENS, _DC), jnp.float32),
        pltpu.SemaphoreType.DMA,
    ],
)
def _sc_gather(idx_hbm, table_hbm, out_hbm, idx_v, tab_v, sem):
    wid = lax.axis_index("c") * _NS + lax.axis_index("s")
    g = wid // _NCH
    c = wid % _NCH
    col0 = c * _DC
    base = g * _ROWS_PER_G

    pltpu.sync_copy(idx_hbm.at[g], idx_v)
    pltpu.sync_copy(table_hbm.at[:, pl.ds(col0, _DC)], tab_v)

    def issue(rb, carry):
        vec = idx_v[pl.ds(rb * 16, 16)]
        for k in range(16):
            v = vec[k]
            dst = out_hbm.at[base + rb * 16 + k, pl.ds(col0, _DC)]
            pltpu.async_copy(tab_v.at[v], dst, sem)
        return carry

    lax.fori_loop(0, _ROWS_PER_G // 16, issue, 0)

    def drain(r, carry):
        pltpu.make_async_copy(
            tab_v.at[0], out_hbm.at[base + r, pl.ds(col0, _DC)], sem
        ).wait()
        return carry

    lax.fori_loop(0, _ROWS_PER_G, drain, 0)


def kernel(prefix, embedding_weight):
    idx = prefix.astype(jnp.int32).T.reshape(_NG, _ROWS_PER_G)
    out = _sc_gather(idx, embedding_weight)
    out = out.reshape(NUM_VIRTUAL_TOKENS, BATCH, EMBED_DIM)
    return out.transpose(1, 0, 2)
